# Initial kernel scaffold; baseline (speedup 1.0000x reference)
import jax
import jax.numpy as jnp
from jax.experimental import pallas as pl


def kernel(x, edge_index, W, attn_l, attn_r):
    raise NotImplementedError("write your pallas kernel here")


# speedup vs baseline: 17.1348x; 17.1348x over previous
import jax
import jax.numpy as jnp
from jax import lax
from jax.experimental import pallas as pl
from jax.experimental.pallas import tpu as pltpu
from jax.experimental.pallas import tpu_sc as plsc

NS = 16
NC = 2
NW = NC * NS
LANES = 16
D = 128
BN = 2000
ZC = 80

_SC_PARAMS = pltpu.CompilerParams(needs_layout_passes=False)


def _project_tc(x, W, attl, attr):
    n = x.shape[0]

    def body(x_ref, w_ref, attl_ref, attr_ref, xp_ref, al_ref, ar_ref):
        xp = lax.dot_general(
            x_ref[...], w_ref[...], (((1,), (1,)), ((), ())),
            preferred_element_type=jnp.float32,
        )
        xp_ref[...] = xp
        al_ref[...] = jnp.sum(xp * attl_ref[...], axis=1, keepdims=True)
        ar_ref[...] = jnp.sum(xp * attr_ref[...], axis=1, keepdims=True)

    return pl.pallas_call(
        body,
        grid=(n // BN,),
        in_specs=[
            pl.BlockSpec((BN, D), lambda i: (i, 0)),
            pl.BlockSpec((D, D), lambda i: (0, 0)),
            pl.BlockSpec((1, D), lambda i: (0, 0)),
            pl.BlockSpec((1, D), lambda i: (0, 0)),
        ],
        out_specs=[
            pl.BlockSpec((BN, D), lambda i: (i, 0)),
            pl.BlockSpec((BN, 1), lambda i: (i, 0)),
            pl.BlockSpec((BN, 1), lambda i: (i, 0)),
        ],
        out_shape=[
            jax.ShapeDtypeStruct((n, D), jnp.float32),
            jax.ShapeDtypeStruct((n, 1), jnp.float32),
            jax.ShapeDtypeStruct((n, 1), jnp.float32),
        ],
    )(x, W, attl, attr)


def _leaky(t):
    return jnp.where(t > 0, t, 0.2 * t)


def _edge_max_sc(al, ar, row3, col3):
    n = al.shape[0]
    nchunk, ch = row3.shape[1], row3.shape[2]
    mesh = plsc.VectorSubcoreMesh(core_axis_name="c", subcore_axis_name="s")

    @pl.kernel(
        out_type=jax.ShapeDtypeStruct((NW * LANES,), jnp.float32),
        mesh=mesh,
        compiler_params=_SC_PARAMS,
        scratch_types=[
            pltpu.VMEM((n,), jnp.float32),
            pltpu.VMEM((n,), jnp.float32),
            pltpu.VMEM((nchunk, ch), jnp.int32),
            pltpu.VMEM((nchunk, ch), jnp.int32),
            pltpu.VMEM((LANES,), jnp.float32),
        ],
    )
    def k(al_hbm, ar_hbm, row_hbm, col_hbm, tmax_hbm,
          al_v, ar_v, row_v, col_v, m_v):
        wid = lax.axis_index("c") * NS + lax.axis_index("s")
        pltpu.sync_copy(al_hbm, al_v)
        pltpu.sync_copy(ar_hbm, ar_v)
        pltpu.sync_copy(row_hbm.at[wid], row_v)
        pltpu.sync_copy(col_hbm.at[wid], col_v)

        def body(j, m):
            for q in range(ch // LANES):
                r16 = row_v[j, pl.ds(q * LANES, LANES)]
                c16 = col_v[j, pl.ds(q * LANES, LANES)]
                t = (plsc.load_gather(al_v, [r16])
                     + plsc.load_gather(ar_v, [c16]))
                m = jnp.maximum(m, _leaky(t))
            return m

        m = lax.fori_loop(
            0, nchunk, body, jnp.full((LANES,), -jnp.inf, jnp.float32))
        m_v[...] = m
        pltpu.sync_copy(m_v, tmax_hbm.at[pl.ds(wid * LANES, LANES)])

    return k(al, ar, row3, col3)


def _accumulate_sc(xp, al, ar, row3, col3, tmax):
    n = al.shape[0]
    nchunk, ch = row3.shape[1], row3.shape[2]
    nzc = n // ZC
    zrounds = (nzc + NS - 1) // NS
    mesh = plsc.VectorSubcoreMesh(core_axis_name="c", subcore_axis_name="s")

    BLK = 8
    nblk, btail = nchunk // BLK, nchunk % BLK

    @pl.kernel(
        out_type=[
            jax.ShapeDtypeStruct((NC, n, D), jnp.float32),
            jax.ShapeDtypeStruct((NC * n,), jnp.float32),
        ],
        mesh=mesh,
        compiler_params=_SC_PARAMS,
        scratch_types=[
            pltpu.VMEM((n,), jnp.float32),
            pltpu.VMEM((n,), jnp.float32),
            pltpu.VMEM((BLK, ch), jnp.int32),
            pltpu.VMEM((BLK, ch), jnp.int32),
            pltpu.VMEM((NW * LANES,), jnp.float32),
            pltpu.VMEM((ch,), jnp.float32),
            pltpu.VMEM((ch, D), jnp.float32),
            pltpu.VMEM((ZC,), jnp.float32),
            pltpu.VMEM((ZC,), jnp.float32),
            pltpu.VMEM_SHARED((n, D), jnp.float32),
            pltpu.VMEM_SHARED((n,), jnp.float32),
        ],
    )
    def k(xp_hbm, al_hbm, ar_hbm, row_hbm, col_hbm, tmax_hbm,
          feat_out, norm_out,
          al_v, ar_v, rb, cb, tm_v, vv, rows, zbn, nvbuf,
          feat_sp, norm_sp):
        c = lax.axis_index("c")
        s = lax.axis_index("s")
        wid = c * NS + s
        pltpu.sync_copy(al_hbm, al_v)
        pltpu.sync_copy(ar_hbm, ar_v)
        pltpu.sync_copy(tmax_hbm, tm_v)

        @pl.loop(0, ZC)
        def _(e):
            for kk in range(D // LANES):
                rows[e, pl.ds(kk * LANES, LANES)] = jnp.zeros(
                    (LANES,), jnp.float32)
        for q in range(ZC // LANES):
            zbn[pl.ds(q * LANES, LANES)] = jnp.zeros((LANES,), jnp.float32)

        for r in range(zrounds):
            cidx = r * NS + s

            @pl.when(cidx < nzc)
            def _():
                pltpu.sync_copy(rows.at[pl.ds(0, ZC)],
                                feat_sp.at[pl.ds(cidx * ZC, ZC)])
                pltpu.sync_copy(zbn, norm_sp.at[pl.ds(cidx * ZC, ZC)])

        m = jnp.full((LANES,), -jnp.inf, jnp.float32)
        for i in range(NW):
            m = jnp.maximum(m, tm_v[pl.ds(i * LANES, LANES)])
        g16 = lax.broadcast_in_dim(
            lax.reduce_max(m, axes=(0,)), (LANES,), ())

        plsc.subcore_barrier()

        def do_chunk(j):
            for q in range(ch // LANES):
                r16 = rb[j, pl.ds(q * LANES, LANES)]
                c16 = cb[j, pl.ds(q * LANES, LANES)]
                t = (plsc.load_gather(al_v, [r16])
                     + plsc.load_gather(ar_v, [c16]))
                u = jnp.exp(_leaky(t) - g16)
                vv[pl.ds(q * LANES, LANES)] = jnp.exp(u)
            pltpu.sync_copy(xp_hbm.at[cb.at[j]], rows)

            @pl.loop(0, ch)
            def _(e):
                e16 = lax.broadcast_in_dim(e, (LANES,), ())
                b16 = plsc.load_gather(vv, [e16])
                for kk in range(D // LANES):
                    sl = pl.ds(kk * LANES, LANES)
                    rows[e, sl] = rows[e, sl] * b16

            pltpu.sync_copy(rows, feat_sp.at[rb.at[j]], add=True)
            pltpu.sync_copy(vv, norm_sp.at[rb.at[j]], add=True)

        for b in range(nblk + (1 if btail else 0)):
            bsz = BLK if b < nblk else btail
            pltpu.sync_copy(row_hbm.at[wid, pl.ds(b * BLK, bsz)],
                            rb.at[pl.ds(0, bsz)])
            pltpu.sync_copy(col_hbm.at[wid, pl.ds(b * BLK, bsz)],
                            cb.at[pl.ds(0, bsz)])

            @pl.loop(0, bsz)
            def _(j):
                do_chunk(j)

        plsc.subcore_barrier()

        for r in range(zrounds):
            cidx = r * NS + s

            @pl.when(cidx < nzc)
            def _():
                pltpu.sync_copy(feat_sp.at[pl.ds(cidx * ZC, ZC)],
                                feat_out.at[c, pl.ds(cidx * ZC, ZC)])
                pltpu.sync_copy(norm_sp.at[pl.ds(cidx * ZC, ZC)], nvbuf)
                pltpu.sync_copy(nvbuf,
                                norm_out.at[pl.ds(c * n + cidx * ZC, ZC)])

    return k(xp, al, ar, row3, col3, tmax)


def _normalize_tc(feat, norm):
    n = feat.shape[1]

    def body(f_ref, z_ref, o_ref):
        fsum = f_ref[0] + f_ref[1]
        zsum = z_ref[0] + z_ref[1]
        o_ref[...] = fsum / (zsum + 1e-16)

    return pl.pallas_call(
        body,
        grid=(n // BN,),
        in_specs=[
            pl.BlockSpec((NC, BN, D), lambda i: (0, i, 0)),
            pl.BlockSpec((NC, BN, 1), lambda i: (0, i, 0)),
        ],
        out_specs=pl.BlockSpec((BN, D), lambda i: (i, 0)),
        out_shape=jax.ShapeDtypeStruct((n, D), jnp.float32),
    )(feat, norm.reshape(NC, n, 1))


def kernel(x, edge_index, W, attn_l, attn_r):
    n, _ = x.shape
    e = edge_index.shape[1]
    ept = e // NW
    ch = 80
    nchunk = ept // ch

    attl = attn_l.reshape(1, D)
    attr = attn_r.reshape(1, D)
    xp, al2, ar2 = _project_tc(x, W, attl, attr)
    al = al2.reshape(n)
    ar = ar2.reshape(n)
    row3 = edge_index[0].reshape(NW, nchunk, ch)
    col3 = edge_index[1].reshape(NW, nchunk, ch)
    tmax = _edge_max_sc(al, ar, row3, col3)
    feat, norm = _accumulate_sc(xp, al, ar, row3, col3, tmax)
    return _normalize_tc(feat, norm)


# speedup vs baseline: 22.4368x; 1.3094x over previous
import jax
import jax.numpy as jnp
from jax import lax
from jax.experimental import pallas as pl
from jax.experimental.pallas import tpu as pltpu
from jax.experimental.pallas import tpu_sc as plsc

NS = 16
NC = 2
NW = NC * NS
LANES = 16
D = 128
BN = 2000
ZC = 80

_SC_PARAMS = pltpu.CompilerParams(needs_layout_passes=False)


def _project_tc(x, W, attl, attr):
    n = x.shape[0]

    def body(x_ref, w_ref, attl_ref, attr_ref, xp_ref, al_ref, ar_ref):
        xp = lax.dot_general(
            x_ref[...], w_ref[...], (((1,), (1,)), ((), ())),
            preferred_element_type=jnp.float32,
        )
        xp_ref[...] = xp
        al_ref[...] = jnp.sum(xp * attl_ref[...], axis=1, keepdims=True)
        ar_ref[...] = jnp.sum(xp * attr_ref[...], axis=1, keepdims=True)

    return pl.pallas_call(
        body,
        grid=(n // BN,),
        in_specs=[
            pl.BlockSpec((BN, D), lambda i: (i, 0)),
            pl.BlockSpec((D, D), lambda i: (0, 0)),
            pl.BlockSpec((1, D), lambda i: (0, 0)),
            pl.BlockSpec((1, D), lambda i: (0, 0)),
        ],
        out_specs=[
            pl.BlockSpec((BN, D), lambda i: (i, 0)),
            pl.BlockSpec((BN, 1), lambda i: (i, 0)),
            pl.BlockSpec((BN, 1), lambda i: (i, 0)),
        ],
        out_shape=[
            jax.ShapeDtypeStruct((n, D), jnp.float32),
            jax.ShapeDtypeStruct((n, 1), jnp.float32),
            jax.ShapeDtypeStruct((n, 1), jnp.float32),
        ],
    )(x, W, attl, attr)


def _leaky(t):
    return jnp.where(t > 0, t, 0.2 * t)


def _edge_max_sc(al, ar, row3, col3):
    n = al.shape[0]
    nchunk, ch = row3.shape[1], row3.shape[2]
    mesh = plsc.VectorSubcoreMesh(core_axis_name="c", subcore_axis_name="s")

    @pl.kernel(
        out_type=jax.ShapeDtypeStruct((NW * LANES,), jnp.float32),
        mesh=mesh,
        compiler_params=_SC_PARAMS,
        scratch_types=[
            pltpu.VMEM((n,), jnp.float32),
            pltpu.VMEM((n,), jnp.float32),
            pltpu.VMEM((nchunk, ch), jnp.int32),
            pltpu.VMEM((nchunk, ch), jnp.int32),
            pltpu.VMEM((LANES,), jnp.float32),
        ],
    )
    def k(al_hbm, ar_hbm, row_hbm, col_hbm, tmax_hbm,
          al_v, ar_v, row_v, col_v, m_v):
        wid = lax.axis_index("c") * NS + lax.axis_index("s")
        pltpu.sync_copy(al_hbm, al_v)
        pltpu.sync_copy(ar_hbm, ar_v)
        pltpu.sync_copy(row_hbm.at[wid], row_v)
        pltpu.sync_copy(col_hbm.at[wid], col_v)

        def body(j, m):
            for q in range(ch // LANES):
                r16 = row_v[j, pl.ds(q * LANES, LANES)]
                c16 = col_v[j, pl.ds(q * LANES, LANES)]
                t = (plsc.load_gather(al_v, [r16])
                     + plsc.load_gather(ar_v, [c16]))
                m = jnp.maximum(m, _leaky(t))
            return m

        m = lax.fori_loop(
            0, nchunk, body, jnp.full((LANES,), -jnp.inf, jnp.float32))
        m_v[...] = m
        pltpu.sync_copy(m_v, tmax_hbm.at[pl.ds(wid * LANES, LANES)])

    return k(al, ar, row3, col3)


def _accumulate_sc(xp, al, ar, row3, col3, tmax):
    n = al.shape[0]
    nchunk, ch = row3.shape[1], row3.shape[2]
    nzc = n // ZC
    zrounds = (nzc + NS - 1) // NS
    mesh = plsc.VectorSubcoreMesh(core_axis_name="c", subcore_axis_name="s")

    BLK = 8
    nblk, btail = nchunk // BLK, nchunk % BLK

    @pl.kernel(
        out_type=[
            jax.ShapeDtypeStruct((NC, n, D), jnp.float32),
            jax.ShapeDtypeStruct((NC * n,), jnp.float32),
        ],
        mesh=mesh,
        compiler_params=_SC_PARAMS,
        scratch_types=[
            pltpu.VMEM((n,), jnp.float32),
            pltpu.VMEM((n,), jnp.float32),
            pltpu.VMEM((BLK, ch), jnp.int32),
            pltpu.VMEM((BLK, ch), jnp.int32),
            pltpu.VMEM((NW * LANES,), jnp.float32),
            pltpu.VMEM((ch,), jnp.float32),
            pltpu.VMEM((ch, D), jnp.float32),
            pltpu.VMEM((ch, D), jnp.float32),
            pltpu.VMEM((ZC,), jnp.float32),
            pltpu.VMEM((ZC,), jnp.float32),
            pltpu.SemaphoreType.DMA,
            pltpu.SemaphoreType.DMA,
            pltpu.VMEM_SHARED((n, D), jnp.float32),
            pltpu.VMEM_SHARED((n,), jnp.float32),
        ],
    )
    def k(xp_hbm, al_hbm, ar_hbm, row_hbm, col_hbm, tmax_hbm,
          feat_out, norm_out,
          al_v, ar_v, rb, cb, tm_v, vv, rows0, rows1, zbn, nvbuf,
          gsem0, gsem1,
          feat_sp, norm_sp):
        c = lax.axis_index("c")
        s = lax.axis_index("s")
        wid = c * NS + s
        pltpu.sync_copy(al_hbm, al_v)
        pltpu.sync_copy(ar_hbm, ar_v)
        pltpu.sync_copy(tmax_hbm, tm_v)

        @pl.loop(0, ZC)
        def _(e):
            for kk in range(D // LANES):
                rows0[e, pl.ds(kk * LANES, LANES)] = jnp.zeros(
                    (LANES,), jnp.float32)
        for q in range(ZC // LANES):
            zbn[pl.ds(q * LANES, LANES)] = jnp.zeros((LANES,), jnp.float32)

        for r in range(zrounds):
            cidx = r * NS + s

            @pl.when(cidx < nzc)
            def _():
                pltpu.sync_copy(rows0.at[pl.ds(0, ZC)],
                                feat_sp.at[pl.ds(cidx * ZC, ZC)])
                pltpu.sync_copy(zbn, norm_sp.at[pl.ds(cidx * ZC, ZC)])

        m = jnp.full((LANES,), -jnp.inf, jnp.float32)
        for i in range(NW):
            m = jnp.maximum(m, tm_v[pl.ds(i * LANES, LANES)])
        g16 = lax.broadcast_in_dim(
            lax.reduce_max(m, axes=(0,)), (LANES,), ())

        plsc.subcore_barrier()

        def process(j, rbuf):
            for q in range(ch // LANES):
                r16 = rb[j, pl.ds(q * LANES, LANES)]
                c16 = cb[j, pl.ds(q * LANES, LANES)]
                t = (plsc.load_gather(al_v, [r16])
                     + plsc.load_gather(ar_v, [c16]))
                u = jnp.exp(_leaky(t) - g16)
                vv[pl.ds(q * LANES, LANES)] = jnp.exp(u)

            @pl.loop(0, ch)
            def _(e):
                e16 = lax.broadcast_in_dim(e, (LANES,), ())
                b16 = plsc.load_gather(vv, [e16])
                for kk in range(D // LANES):
                    sl = pl.ds(kk * LANES, LANES)
                    rbuf[e, sl] = rbuf[e, sl] * b16

            pltpu.sync_copy(rbuf, feat_sp.at[rb.at[j]], add=True)
            pltpu.sync_copy(vv, norm_sp.at[rb.at[j]], add=True)

        ngrp = BLK // 2
        for b in range(nblk + (1 if btail else 0)):
            bsz = BLK if b < nblk else btail
            pltpu.sync_copy(row_hbm.at[wid, pl.ds(b * BLK, bsz)],
                            rb.at[pl.ds(0, bsz)])
            pltpu.sync_copy(col_hbm.at[wid, pl.ds(b * BLK, bsz)],
                            cb.at[pl.ds(0, bsz)])

            if bsz == BLK:
                pltpu.async_copy(xp_hbm.at[cb.at[0]], rows0, gsem0)

                @pl.loop(0, ngrp)
                def _(g):
                    j0 = 2 * g
                    pltpu.make_async_copy(
                        xp_hbm.at[cb.at[j0]], rows0, gsem0).wait()
                    pltpu.async_copy(xp_hbm.at[cb.at[j0 + 1]], rows1, gsem1)
                    process(j0, rows0)
                    pltpu.make_async_copy(
                        xp_hbm.at[cb.at[j0 + 1]], rows1, gsem1).wait()

                    @pl.when(g + 1 < ngrp)
                    def _():
                        pltpu.async_copy(
                            xp_hbm.at[cb.at[j0 + 2]], rows0, gsem0)

                    process(j0 + 1, rows1)
            else:
                @pl.loop(0, bsz)
                def _(j):
                    pltpu.sync_copy(xp_hbm.at[cb.at[j]], rows0)
                    process(j, rows0)

        plsc.subcore_barrier()

        for r in range(zrounds):
            cidx = r * NS + s

            @pl.when(cidx < nzc)
            def _():
                pltpu.sync_copy(feat_sp.at[pl.ds(cidx * ZC, ZC)],
                                feat_out.at[c, pl.ds(cidx * ZC, ZC)])
                pltpu.sync_copy(norm_sp.at[pl.ds(cidx * ZC, ZC)], nvbuf)
                pltpu.sync_copy(nvbuf,
                                norm_out.at[pl.ds(c * n + cidx * ZC, ZC)])

    return k(xp, al, ar, row3, col3, tmax)


def _normalize_tc(feat, norm):
    n = feat.shape[1]

    def body(f_ref, z_ref, o_ref):
        fsum = f_ref[0] + f_ref[1]
        zsum = z_ref[0] + z_ref[1]
        o_ref[...] = fsum / (zsum + 1e-16)

    return pl.pallas_call(
        body,
        grid=(n // BN,),
        in_specs=[
            pl.BlockSpec((NC, BN, D), lambda i: (0, i, 0)),
            pl.BlockSpec((NC, BN, 1), lambda i: (0, i, 0)),
        ],
        out_specs=pl.BlockSpec((BN, D), lambda i: (i, 0)),
        out_shape=jax.ShapeDtypeStruct((n, D), jnp.float32),
    )(feat, norm.reshape(NC, n, 1))


def kernel(x, edge_index, W, attn_l, attn_r):
    n, _ = x.shape
    e = edge_index.shape[1]
    ept = e // NW
    ch = 80
    nchunk = ept // ch

    attl = attn_l.reshape(1, D)
    attr = attn_r.reshape(1, D)
    xp, al2, ar2 = _project_tc(x, W, attl, attr)
    al = al2.reshape(n)
    ar = ar2.reshape(n)
    row3 = edge_index[0].reshape(NW, nchunk, ch)
    col3 = edge_index[1].reshape(NW, nchunk, ch)
    tmax = _edge_max_sc(al, ar, row3, col3)
    feat, norm = _accumulate_sc(xp, al, ar, row3, col3, tmax)
    return _normalize_tc(feat, norm)


# speedup vs baseline: 22.5018x; 1.0029x over previous
import jax
import jax.numpy as jnp
from jax import lax
from jax.experimental import pallas as pl
from jax.experimental.pallas import tpu as pltpu
from jax.experimental.pallas import tpu_sc as plsc

NS = 16
NC = 2
NW = NC * NS
LANES = 16
D = 128
BN = 2000
ZC = 80

_SC_PARAMS = pltpu.CompilerParams(needs_layout_passes=False)


def _project_tc(x, W, attl, attr):
    n = x.shape[0]

    def body(x_ref, w_ref, attl_ref, attr_ref, xp_ref, al_ref, ar_ref):
        xp = lax.dot_general(
            x_ref[...], w_ref[...], (((1,), (1,)), ((), ())),
            preferred_element_type=jnp.float32,
        )
        xp_ref[...] = xp
        al_ref[...] = jnp.sum(xp * attl_ref[...], axis=1, keepdims=True)
        ar_ref[...] = jnp.sum(xp * attr_ref[...], axis=1, keepdims=True)

    return pl.pallas_call(
        body,
        grid=(n // BN,),
        in_specs=[
            pl.BlockSpec((BN, D), lambda i: (i, 0)),
            pl.BlockSpec((D, D), lambda i: (0, 0)),
            pl.BlockSpec((1, D), lambda i: (0, 0)),
            pl.BlockSpec((1, D), lambda i: (0, 0)),
        ],
        out_specs=[
            pl.BlockSpec((BN, D), lambda i: (i, 0)),
            pl.BlockSpec((BN, 1), lambda i: (i, 0)),
            pl.BlockSpec((BN, 1), lambda i: (i, 0)),
        ],
        out_shape=[
            jax.ShapeDtypeStruct((n, D), jnp.float32),
            jax.ShapeDtypeStruct((n, 1), jnp.float32),
            jax.ShapeDtypeStruct((n, 1), jnp.float32),
        ],
    )(x, W, attl, attr)


def _leaky(t):
    return jnp.where(t > 0, t, 0.2 * t)


def _edge_max_sc(al, ar, row3, col3):
    n = al.shape[0]
    nchunk, ch = row3.shape[1], row3.shape[2]
    mesh = plsc.VectorSubcoreMesh(core_axis_name="c", subcore_axis_name="s")

    @pl.kernel(
        out_type=jax.ShapeDtypeStruct((NW * LANES,), jnp.float32),
        mesh=mesh,
        compiler_params=_SC_PARAMS,
        scratch_types=[
            pltpu.VMEM((n,), jnp.float32),
            pltpu.VMEM((n,), jnp.float32),
            pltpu.VMEM((nchunk, ch), jnp.int32),
            pltpu.VMEM((nchunk, ch), jnp.int32),
            pltpu.VMEM((LANES,), jnp.float32),
        ],
    )
    def k(al_hbm, ar_hbm, row_hbm, col_hbm, tmax_hbm,
          al_v, ar_v, row_v, col_v, m_v):
        wid = lax.axis_index("c") * NS + lax.axis_index("s")
        pltpu.sync_copy(al_hbm, al_v)
        pltpu.sync_copy(ar_hbm, ar_v)
        pltpu.sync_copy(row_hbm.at[wid], row_v)
        pltpu.sync_copy(col_hbm.at[wid], col_v)

        def body(j, m):
            for q in range(ch // LANES):
                r16 = row_v[j, pl.ds(q * LANES, LANES)]
                c16 = col_v[j, pl.ds(q * LANES, LANES)]
                t = (plsc.load_gather(al_v, [r16])
                     + plsc.load_gather(ar_v, [c16]))
                m = jnp.maximum(m, _leaky(t))
            return m

        m = lax.fori_loop(
            0, nchunk, body, jnp.full((LANES,), -jnp.inf, jnp.float32))
        m_v[...] = m
        pltpu.sync_copy(m_v, tmax_hbm.at[pl.ds(wid * LANES, LANES)])

    return k(al, ar, row3, col3)


def _accumulate_sc(xp, al, ar, row3, col3, tmax):
    n = al.shape[0]
    nchunk, ch = row3.shape[1], row3.shape[2]
    nzc = n // ZC
    zrounds = (nzc + NS - 1) // NS
    mesh = plsc.VectorSubcoreMesh(core_axis_name="c", subcore_axis_name="s")

    BLK = 8
    nblk, btail = nchunk // BLK, nchunk % BLK

    @pl.kernel(
        out_type=[
            jax.ShapeDtypeStruct((NC, n, D), jnp.float32),
            jax.ShapeDtypeStruct((NC * n,), jnp.float32),
        ],
        mesh=mesh,
        compiler_params=_SC_PARAMS,
        scratch_types=[
            pltpu.VMEM((n,), jnp.float32),
            pltpu.VMEM((n,), jnp.float32),
            pltpu.VMEM((BLK, ch), jnp.int32),
            pltpu.VMEM((BLK, ch), jnp.int32),
            pltpu.VMEM((NW * LANES,), jnp.float32),
            pltpu.VMEM((ch,), jnp.float32),
            pltpu.VMEM((ch, D), jnp.float32),
            pltpu.VMEM((ch, D), jnp.float32),
            pltpu.VMEM((ZC,), jnp.float32),
            pltpu.VMEM((ZC,), jnp.float32),
            pltpu.SemaphoreType.DMA,
            pltpu.SemaphoreType.DMA,
            pltpu.SemaphoreType.DMA,
            pltpu.VMEM_SHARED((n, D), jnp.float32),
            pltpu.VMEM_SHARED((n,), jnp.float32),
        ],
    )
    def k(xp_hbm, al_hbm, ar_hbm, row_hbm, col_hbm, tmax_hbm,
          feat_out, norm_out,
          al_v, ar_v, rb, cb, tm_v, vv, rows0, rows1, zbn, nvbuf,
          gsem0, gsem1, zsem,
          feat_sp, norm_sp):
        c = lax.axis_index("c")
        s = lax.axis_index("s")
        wid = c * NS + s
        pltpu.sync_copy(al_hbm, al_v)
        pltpu.sync_copy(ar_hbm, ar_v)
        pltpu.sync_copy(tmax_hbm, tm_v)

        @pl.loop(0, ZC)
        def _(e):
            for kk in range(D // LANES):
                rows0[e, pl.ds(kk * LANES, LANES)] = jnp.zeros(
                    (LANES,), jnp.float32)
        for q in range(ZC // LANES):
            zbn[pl.ds(q * LANES, LANES)] = jnp.zeros((LANES,), jnp.float32)

        for r in range(zrounds):
            cidx = r * NS + s

            @pl.when(cidx < nzc)
            def _():
                pltpu.async_copy(rows0.at[pl.ds(0, ZC)],
                                 feat_sp.at[pl.ds(cidx * ZC, ZC)], zsem)
                pltpu.async_copy(zbn, norm_sp.at[pl.ds(cidx * ZC, ZC)],
                                 zsem)

        for r in range(zrounds):
            cidx = r * NS + s

            @pl.when(cidx < nzc)
            def _():
                pltpu.make_async_copy(
                    rows0.at[pl.ds(0, ZC)],
                    feat_sp.at[pl.ds(cidx * ZC, ZC)], zsem).wait()
                pltpu.make_async_copy(
                    zbn, norm_sp.at[pl.ds(cidx * ZC, ZC)], zsem).wait()

        m = jnp.full((LANES,), -jnp.inf, jnp.float32)
        for i in range(NW):
            m = jnp.maximum(m, tm_v[pl.ds(i * LANES, LANES)])
        g16 = lax.broadcast_in_dim(
            lax.reduce_max(m, axes=(0,)), (LANES,), ())

        plsc.subcore_barrier()

        def process(j, rbuf):
            for q in range(ch // LANES):
                r16 = rb[j, pl.ds(q * LANES, LANES)]
                c16 = cb[j, pl.ds(q * LANES, LANES)]
                t = (plsc.load_gather(al_v, [r16])
                     + plsc.load_gather(ar_v, [c16]))
                u = jnp.exp(_leaky(t) - g16)
                vv[pl.ds(q * LANES, LANES)] = jnp.exp(u)

            @pl.loop(0, ch)
            def _(e):
                e16 = lax.broadcast_in_dim(e, (LANES,), ())
                b16 = plsc.load_gather(vv, [e16])
                for kk in range(D // LANES):
                    sl = pl.ds(kk * LANES, LANES)
                    rbuf[e, sl] = rbuf[e, sl] * b16

            pltpu.sync_copy(rbuf, feat_sp.at[rb.at[j]], add=True)
            pltpu.sync_copy(vv, norm_sp.at[rb.at[j]], add=True)

        ngrp = BLK // 2
        for b in range(nblk + (1 if btail else 0)):
            bsz = BLK if b < nblk else btail
            pltpu.sync_copy(row_hbm.at[wid, pl.ds(b * BLK, bsz)],
                            rb.at[pl.ds(0, bsz)])
            pltpu.sync_copy(col_hbm.at[wid, pl.ds(b * BLK, bsz)],
                            cb.at[pl.ds(0, bsz)])

            if bsz == BLK:
                pltpu.async_copy(xp_hbm.at[cb.at[0]], rows0, gsem0)

                @pl.loop(0, ngrp)
                def _(g):
                    j0 = 2 * g
                    pltpu.make_async_copy(
                        xp_hbm.at[cb.at[j0]], rows0, gsem0).wait()
                    pltpu.async_copy(xp_hbm.at[cb.at[j0 + 1]], rows1, gsem1)
                    process(j0, rows0)
                    pltpu.make_async_copy(
                        xp_hbm.at[cb.at[j0 + 1]], rows1, gsem1).wait()

                    @pl.when(g + 1 < ngrp)
                    def _():
                        pltpu.async_copy(
                            xp_hbm.at[cb.at[j0 + 2]], rows0, gsem0)

                    process(j0 + 1, rows1)
            else:
                @pl.loop(0, bsz)
                def _(j):
                    pltpu.sync_copy(xp_hbm.at[cb.at[j]], rows0)
                    process(j, rows0)

        plsc.subcore_barrier()

        for r in range(zrounds):
            cidx = r * NS + s

            @pl.when(cidx < nzc)
            def _():
                pltpu.sync_copy(feat_sp.at[pl.ds(cidx * ZC, ZC)],
                                feat_out.at[c, pl.ds(cidx * ZC, ZC)])
                pltpu.sync_copy(norm_sp.at[pl.ds(cidx * ZC, ZC)], nvbuf)
                pltpu.sync_copy(nvbuf,
                                norm_out.at[pl.ds(c * n + cidx * ZC, ZC)])

    return k(xp, al, ar, row3, col3, tmax)


def _normalize_tc(feat, norm):
    n = feat.shape[1]

    def body(f_ref, z_ref, o_ref):
        fsum = f_ref[0] + f_ref[1]
        zsum = z_ref[0] + z_ref[1]
        o_ref[...] = fsum / (zsum + 1e-16)

    return pl.pallas_call(
        body,
        grid=(n // BN,),
        in_specs=[
            pl.BlockSpec((NC, BN, D), lambda i: (0, i, 0)),
            pl.BlockSpec((NC, BN, 1), lambda i: (0, i, 0)),
        ],
        out_specs=pl.BlockSpec((BN, D), lambda i: (i, 0)),
        out_shape=jax.ShapeDtypeStruct((n, D), jnp.float32),
    )(feat, norm.reshape(NC, n, 1))


def kernel(x, edge_index, W, attn_l, attn_r):
    n, _ = x.shape
    e = edge_index.shape[1]
    ept = e // NW
    ch = 80
    nchunk = ept // ch

    attl = attn_l.reshape(1, D)
    attr = attn_r.reshape(1, D)
    xp, al2, ar2 = _project_tc(x, W, attl, attr)
    al = al2.reshape(n)
    ar = ar2.reshape(n)
    row3 = edge_index[0].reshape(NW, nchunk, ch)
    col3 = edge_index[1].reshape(NW, nchunk, ch)
    tmax = _edge_max_sc(al, ar, row3, col3)
    feat, norm = _accumulate_sc(xp, al, ar, row3, col3, tmax)
    return _normalize_tc(feat, norm)
